# baseline (speedup 1.0000x reference)
import functools

import jax
import jax.numpy as jnp
from jax import lax
from jax.experimental import pallas as pl
from jax.experimental.pallas import tpu as pltpu
from jax.experimental.pallas import tpu_sc as plsc

_EMBED_DIM = 32
_FC_DIM = 200
_BATCH = 16384

_NC = 2
_NS = 16
_NW = _NC * _NS
_B_PER_W = _BATCH // _NW
_UVECS = _B_PER_W // 16


def _make_gather():
    mesh = plsc.VectorSubcoreMesh(
        core_axis_name="c", subcore_axis_name="s",
        num_cores=_NC, num_subcores=_NS)

    @functools.partial(
        pl.kernel,
        mesh=mesh,
        out_type=jax.ShapeDtypeStruct((_EMBED_DIM, _BATCH), jnp.float32),
        scratch_types=[
            pltpu.VMEM((_B_PER_W,), jnp.int32),
            pltpu.VMEM((_EMBED_DIM, 16 * 128), jnp.float32),
            pltpu.VMEM((_EMBED_DIM, _B_PER_W), jnp.float32),
            pltpu.SemaphoreType.DMA((16,)),
        ],
        compiler_params=pltpu.CompilerParams(
            needs_layout_passes=False,
            use_tc_tiling_on_sc=True),
    )
    def gather(idx_hbm, tableT_hbm, out_hbm, idx_v, stage_v, col_v, sems):
        wid = lax.axis_index("s") * _NC + lax.axis_index("c")
        base = wid * _B_PER_W
        pltpu.sync_copy(idx_hbm.at[pl.ds(base, _B_PER_W)], idx_v)

        lanes = lax.iota(jnp.int32, 16)

        def issue(u, l):
            cb = pl.multiple_of(u - lax.bitwise_and(u, 127), 128)
            for g in range(4):
                pltpu.async_copy(
                    tableT_hbm.at[pl.ds(g * 8, 8), pl.ds(cb, 128)],
                    stage_v.at[pl.ds(g * 8, 8), pl.ds(l * 128, 128)],
                    sems.at[l])

        vec0 = idx_v[pl.ds(0, 16)]
        for l in range(16):
            issue(vec0[l], l)

        @pl.loop(0, _UVECS)
        def group(i):
            vec = idx_v[pl.ds(i * 16, 16)]
            nxt = jnp.minimum(i + 1, _UVECS - 1)
            vec_n = idx_v[pl.ds(nxt * 16, 16)]
            for l in range(16):
                pltpu.make_async_copy(
                    tableT_hbm.at[:, pl.ds(0, 128)],
                    stage_v.at[:, pl.ds(l * 128, 128)], sems.at[l]).wait()
                u = vec[l]
                colb = jnp.broadcast_to(
                    l * 128 + lax.bitwise_and(u, 127), (16,))
                v1 = plsc.load_gather(stage_v, [lanes, colb])
                v2 = plsc.load_gather(stage_v, [lanes + 16, colb])
                cpos = jnp.broadcast_to(i * 16 + l, (16,))
                plsc.store_scatter(col_v, [lanes, cpos], v1)
                plsc.store_scatter(col_v, [lanes + 16, cpos], v2)

                @pl.when(i < _UVECS - 1)
                def _():
                    issue(vec_n[l], l)

        pltpu.sync_copy(col_v, out_hbm.at[:, pl.ds(base, _B_PER_W)])

    return gather


_gather = _make_gather()


def _fc_body(wt_ref, embT_ref, b_ref, outT_ref):
    acc = jnp.dot(wt_ref[...], embT_ref[...],
                  preferred_element_type=jnp.float32)
    outT_ref[...] = jnp.maximum(acc + b_ref[...], 0.0)


def _fc(Wt, embT, b2d):
    blk = 8192
    return pl.pallas_call(
        _fc_body,
        grid=(_BATCH // blk,),
        in_specs=[
            pl.BlockSpec((_FC_DIM, _EMBED_DIM), lambda i: (0, 0)),
            pl.BlockSpec((_EMBED_DIM, blk), lambda i: (0, i)),
            pl.BlockSpec((_FC_DIM, 1), lambda i: (0, 0)),
        ],
        out_specs=pl.BlockSpec((_FC_DIM, blk), lambda i: (0, i)),
        out_shape=jax.ShapeDtypeStruct((_FC_DIM, _BATCH), jnp.float32),
    )(Wt, embT, b2d)


def kernel(indices, table, W, b):
    idx = indices.reshape(_BATCH).astype(jnp.int32)
    embT = _gather(idx, table.T)
    outT = _fc(W.T, embT, b.reshape(_FC_DIM, 1))
    return outT.T.reshape(_BATCH, 1, _FC_DIM)
